# Initial kernel scaffold; baseline (speedup 1.0000x reference)
import jax
import jax.numpy as jnp
from jax import lax
from jax.experimental import pallas as pl
from jax.experimental.pallas import tpu as pltpu
from jax.experimental.pallas import tpu_sc as plsc

NC = 2
NS = 16
NW = NC * NS
CH = 128


def _make_body(B, H, D):
  n_chunks_nodes = B // (NW * CH)
  n_chunks_neigh = (B * H) // (NW * CH)
  rows_n = n_chunks_nodes * CH
  rows_e = n_chunks_neigh * CH

  def body(nodes_hbm, neigh_hbm, user_hbm, item_hbm,
           node_out, neigh_out,
           idx_n_v, idx_e_v, rows_v, gsem):
    w = lax.axis_index("s") * NC + lax.axis_index("c")

    pltpu.sync_copy(nodes_hbm.at[w], idx_n_v)
    pltpu.sync_copy(neigh_hbm.at[w], idx_e_v)

    nbase = w * rows_n
    ebase = w * rows_e

    for j in range(n_chunks_nodes):
      pltpu.async_copy(user_hbm.at[idx_n_v.at[j]], rows_v, gsem).wait()
      pltpu.sync_copy(rows_v, node_out.at[pl.ds(nbase + j * CH, CH)])

    @pl.loop(0, n_chunks_neigh)
    def _chunk(c):
      pltpu.async_copy(item_hbm.at[idx_e_v.at[c]], rows_v, gsem).wait()
      pltpu.sync_copy(rows_v, neigh_out.at[pl.ds(ebase + c * CH, CH)])

  return body, n_chunks_nodes, n_chunks_neigh


def kernel(nodes, neighbors, degrees, user_table, item_table):
  B, H = neighbors.shape
  D = user_table.shape[1]
  assert B % (NW * CH) == 0 and (B * H) % (NW * CH) == 0

  body, ncn, nce = _make_body(B, H, D)

  mesh = plsc.VectorSubcoreMesh(
      core_axis_name="c", subcore_axis_name="s",
      num_cores=NC, num_subcores=NS)

  run = pl.kernel(
      body,
      out_type=(
          jax.ShapeDtypeStruct((B, D), user_table.dtype),
          jax.ShapeDtypeStruct((B * H, D), item_table.dtype),
      ),
      mesh=mesh,
      scratch_types=[
          pltpu.VMEM((ncn, CH), jnp.int32),
          pltpu.VMEM((nce, CH), jnp.int32),
          pltpu.VMEM((CH, D), jnp.float32),
          pltpu.SemaphoreType.DMA,
      ],
  )

  nodes_r = nodes.astype(jnp.int32).reshape(NW, ncn, CH)
  neigh_r = neighbors.astype(jnp.int32).reshape(NW, nce, CH)
  node_emb, neigh_flat = run(nodes_r, neigh_r, user_table, item_table)
  return (node_emb, neigh_flat.reshape(B, H, D), degrees)


# speedup vs baseline: 1.2985x; 1.2985x over previous
import jax
import jax.numpy as jnp
from jax import lax
from jax.experimental import pallas as pl
from jax.experimental.pallas import tpu as pltpu
from jax.experimental.pallas import tpu_sc as plsc

NC = 2
NS = 16
NW = NC * NS
CH = 128


def _make_body(B, H, D):
  n_chunks_nodes = B // (NW * CH)
  n_chunks_neigh = (B * H) // (NW * CH)
  rows_n = n_chunks_nodes * CH
  rows_e = n_chunks_neigh * CH

  def body(nodes_hbm, neigh_hbm, user_hbm, item_hbm,
           node_out, neigh_out,
           idx_n_v, idx_e_v, rows_v, gsem):
    w = lax.axis_index("s") * NC + lax.axis_index("c")

    pltpu.sync_copy(nodes_hbm.at[w], idx_n_v)
    pltpu.sync_copy(neigh_hbm.at[w], idx_e_v)

    nbase = w * rows_n
    ebase = w * rows_e

    for j in range(n_chunks_nodes):
      pltpu.async_copy(user_hbm.at[idx_n_v.at[j]], rows_v, gsem).wait()
      pltpu.sync_copy(rows_v, node_out.at[pl.ds(nbase + j * CH, CH)])

    @pl.loop(0, n_chunks_neigh)
    def _chunk(c):
      pltpu.async_copy(item_hbm.at[idx_e_v.at[c]], rows_v, gsem).wait()
      pltpu.sync_copy(rows_v, neigh_out.at[pl.ds(ebase + c * CH, CH)])

  return body, n_chunks_nodes, n_chunks_neigh


def kernel(nodes, neighbors, degrees, user_table, item_table):
  B, H = neighbors.shape
  D = user_table.shape[1]
  assert B % (NW * CH) == 0 and (B * H) % (NW * CH) == 0

  body, ncn, nce = _make_body(B, H, D)

  mesh = plsc.VectorSubcoreMesh(
      core_axis_name="c", subcore_axis_name="s",
      num_cores=NC, num_subcores=NS)

  run = pl.kernel(
      body,
      out_type=(
          jax.ShapeDtypeStruct((B, D), user_table.dtype),
          jax.ShapeDtypeStruct((B * H, D), item_table.dtype),
      ),
      mesh=mesh,
      compiler_params=pltpu.CompilerParams(use_tc_tiling_on_sc=False),
      scratch_types=[
          pltpu.VMEM((ncn, CH), jnp.int32),
          pltpu.VMEM((nce, CH), jnp.int32),
          pltpu.VMEM((CH, D), jnp.float32),
          pltpu.SemaphoreType.DMA,
      ],
  )

  nodes_r = nodes.astype(jnp.int32).reshape(NW, ncn, CH)
  neigh_r = neighbors.astype(jnp.int32).reshape(NW, nce, CH)
  node_emb, neigh_flat = run(nodes_r, neigh_r, user_table, item_table)
  return (node_emb, neigh_flat.reshape(B, H, D), degrees)


# speedup vs baseline: 1.4039x; 1.0812x over previous
import jax
import jax.numpy as jnp
from jax import lax
from jax.experimental import pallas as pl
from jax.experimental.pallas import tpu as pltpu
from jax.experimental.pallas import tpu_sc as plsc

NC = 2
NS = 16
NW = NC * NS
CH = 128
NBUF = 8


def _make_body(B, H, D):
  ncn = B // (NW * CH)
  nce = (B * H) // (NW * CH)
  assert nce % NBUF == 0
  rows_n = ncn * CH
  rows_e = nce * CH

  def body(nodes_hbm, neigh_hbm, user_hbm, item_hbm,
           node_out, neigh_out, idx_n_v, idx_e_v, *scratch):
    bufs = scratch[:NBUF]
    gsem = scratch[NBUF:2 * NBUF]
    ssem = scratch[2 * NBUF:3 * NBUF]

    w = lax.axis_index("s") * NC + lax.axis_index("c")
    pltpu.sync_copy(nodes_hbm.at[w], idx_n_v)
    pltpu.sync_copy(neigh_hbm.at[w], idx_e_v)
    nbase = w * rows_n
    ebase = w * rows_e

    for j in range(ncn):
      pltpu.async_copy(user_hbm.at[idx_n_v.at[j]], bufs[j % NBUF],
                       gsem[j % NBUF])
    for j in range(ncn):
      b = j % NBUF
      pltpu.make_async_copy(user_hbm.at[idx_n_v.at[j]], bufs[b],
                            gsem[b]).wait()
      pltpu.async_copy(bufs[b], node_out.at[pl.ds(nbase + j * CH, CH)],
                       ssem[b])
    for j in range(ncn):
      b = j % NBUF
      pltpu.make_async_copy(bufs[b], node_out.at[pl.ds(nbase + j * CH, CH)],
                            ssem[b]).wait()

    for b in range(NBUF):
      pltpu.async_copy(item_hbm.at[idx_e_v.at[b]], bufs[b], gsem[b])

    def _do_round(c0, refill):
      for b in range(NBUF):
        c = c0 + b
        pltpu.make_async_copy(item_hbm.at[idx_e_v.at[c]], bufs[b],
                              gsem[b]).wait()
        pltpu.async_copy(bufs[b], neigh_out.at[pl.ds(ebase + c * CH, CH)],
                         ssem[b])
      for b in range(NBUF):
        c = c0 + b
        pltpu.make_async_copy(bufs[b],
                              neigh_out.at[pl.ds(ebase + c * CH, CH)],
                              ssem[b]).wait()
        if refill:
          pltpu.async_copy(item_hbm.at[idx_e_v.at[c + NBUF]], bufs[b],
                           gsem[b])

    @pl.loop(0, nce - NBUF, step=NBUF)
    def _round(c0):
      _do_round(c0, True)

    _do_round(nce - NBUF, False)

  return body, ncn, nce


def kernel(nodes, neighbors, degrees, user_table, item_table):
  B, H = neighbors.shape
  D = user_table.shape[1]
  assert B % (NW * CH) == 0 and (B * H) % (NW * CH) == 0

  body, ncn, nce = _make_body(B, H, D)

  mesh = plsc.VectorSubcoreMesh(
      core_axis_name="c", subcore_axis_name="s",
      num_cores=NC, num_subcores=NS)

  scratch = ([pltpu.VMEM((ncn, CH), jnp.int32),
              pltpu.VMEM((nce, CH), jnp.int32)]
             + [pltpu.VMEM((CH, D), jnp.float32) for _ in range(NBUF)]
             + [pltpu.SemaphoreType.DMA for _ in range(2 * NBUF)])

  run = pl.kernel(
      body,
      out_type=(
          jax.ShapeDtypeStruct((B, D), user_table.dtype),
          jax.ShapeDtypeStruct((B * H, D), item_table.dtype),
      ),
      mesh=mesh,
      compiler_params=pltpu.CompilerParams(use_tc_tiling_on_sc=False),
      scratch_types=scratch,
  )

  nodes_r = nodes.astype(jnp.int32).reshape(NW, ncn, CH)
  neigh_r = neighbors.astype(jnp.int32).reshape(NW, nce, CH)
  node_emb, neigh_flat = run(nodes_r, neigh_r, user_table, item_table)
  return (node_emb, neigh_flat.reshape(B, H, D), degrees)
